# baseline (speedup 1.0000x reference)
import functools

import jax
import jax.numpy as jnp
from jax.experimental import pallas as pl
from jax.experimental.pallas import tpu as pltpu

_LN_EPS = 1e-5
_ROWS_PER_BLOCK = 512


def _fused_body(seq_ref, msa_ref, gw_ref, gamma_ref, beta_ref, gb_ref, out_ref):
    g = gamma_ref[...]
    bt = beta_ref[...]
    w = gw_ref[...]
    dws = g[0:1, :] * (w[0:1, :] - w[2:3, :])
    dwm = g[1:2, :] * (w[1:2, :] - w[3:4, :])
    da = jnp.sum(dws, axis=-1, keepdims=True) + jnp.sum(dwm, axis=-1, keepdims=True)
    dcb = (jnp.sum(bt[0:1, :] * (w[0:1, :] - w[2:3, :]), axis=-1, keepdims=True)
           + jnp.sum(bt[1:2, :] * (w[1:2, :] - w[3:4, :]), axis=-1, keepdims=True)
           + (gb_ref[0:1, 0:1] - gb_ref[0:1, 1:2]))

    s = seq_ref[...]
    m = msa_ref[...]
    n_inv = 1.0 / (2.0 * s.shape[-1])
    row_sum = (jnp.sum(s, axis=-1, keepdims=True)
               + jnp.sum(m, axis=-1, keepdims=True))
    row_sq = (jnp.sum(s * s, axis=-1, keepdims=True)
              + jnp.sum(m * m, axis=-1, keepdims=True))
    mean = row_sum * n_inv
    var = row_sq * n_inv - mean * mean
    rstd = jax.lax.rsqrt(var + _LN_EPS)
    t = (jnp.sum(s * dws, axis=-1, keepdims=True)
         + jnp.sum(m * dwm, axis=-1, keepdims=True))
    dl = (t - mean * da) * rstd + dcb
    w0 = jax.nn.sigmoid(dl)
    out_ref[...] = m + w0 * (s - m)


@jax.jit
def kernel(seq_feat, msa_feat, ln_gamma, ln_beta, gate_w, gate_b):
    B, S, D = seq_feat.shape
    rows = B * S
    seq2 = seq_feat.reshape(rows, D)
    msa2 = msa_feat.reshape(rows, D)
    gamma2 = ln_gamma.reshape(2, D)
    beta2 = ln_beta.reshape(2, D)
    gw = gate_w.reshape(4, D)
    gb = gate_b.reshape(1, 2)

    nblk = rows // _ROWS_PER_BLOCK
    row_spec = pl.BlockSpec((_ROWS_PER_BLOCK, D), lambda i: (i, 0))
    full = lambda shape: pl.BlockSpec(shape, lambda i: (0,) * len(shape))

    out = pl.pallas_call(
        _fused_body,
        out_shape=jax.ShapeDtypeStruct((rows, D), seq_feat.dtype),
        grid=(nblk,),
        in_specs=[
            row_spec,
            row_spec,
            full((4, D)),
            full((2, D)),
            full((2, D)),
            full((1, 2)),
        ],
        out_specs=row_spec,
        compiler_params=pltpu.CompilerParams(
            dimension_semantics=("parallel",),
        ),
        name="attention_fusion",
    )(seq2, msa2, gw, gamma2, beta2, gb)
    return out.reshape(B, S, D)


# speedup vs baseline: 1.1463x; 1.1463x over previous
import functools

import jax
import jax.numpy as jnp
from jax.experimental import pallas as pl
from jax.experimental.pallas import tpu as pltpu

_LN_EPS = 1e-5
_ROWS_PER_BLOCK = 2048


def _fused_body(seq_ref, msa_ref, gw_ref, gamma_ref, beta_ref, gb_ref, out_ref):
    g = gamma_ref[...]
    bt = beta_ref[...]
    w = gw_ref[...]
    dws = g[0:1, :] * (w[0:1, :] - w[2:3, :])
    dwm = g[1:2, :] * (w[1:2, :] - w[3:4, :])
    da = jnp.sum(dws, axis=-1, keepdims=True) + jnp.sum(dwm, axis=-1, keepdims=True)
    dcb = (jnp.sum(bt[0:1, :] * (w[0:1, :] - w[2:3, :]), axis=-1, keepdims=True)
           + jnp.sum(bt[1:2, :] * (w[1:2, :] - w[3:4, :]), axis=-1, keepdims=True)
           + (gb_ref[0:1, 0:1] - gb_ref[0:1, 1:2]))

    s = seq_ref[...]
    m = msa_ref[...]
    n_inv = 1.0 / (2.0 * s.shape[-1])
    row_sum = (jnp.sum(s, axis=-1, keepdims=True)
               + jnp.sum(m, axis=-1, keepdims=True))
    row_sq = (jnp.sum(s * s, axis=-1, keepdims=True)
              + jnp.sum(m * m, axis=-1, keepdims=True))
    mean = row_sum * n_inv
    var = row_sq * n_inv - mean * mean
    rstd = jax.lax.rsqrt(var + _LN_EPS)
    t = (jnp.sum(s * dws, axis=-1, keepdims=True)
         + jnp.sum(m * dwm, axis=-1, keepdims=True))
    dl = (t - mean * da) * rstd + dcb
    w0 = jax.nn.sigmoid(dl)
    out_ref[...] = m + w0 * (s - m)


@jax.jit
def kernel(seq_feat, msa_feat, ln_gamma, ln_beta, gate_w, gate_b):
    B, S, D = seq_feat.shape
    rows = B * S
    seq2 = seq_feat.reshape(rows, D)
    msa2 = msa_feat.reshape(rows, D)
    gamma2 = ln_gamma.reshape(2, D)
    beta2 = ln_beta.reshape(2, D)
    gw = gate_w.reshape(4, D)
    gb = gate_b.reshape(1, 2)

    nblk = rows // _ROWS_PER_BLOCK
    row_spec = pl.BlockSpec((_ROWS_PER_BLOCK, D), lambda i: (i, 0))
    full = lambda shape: pl.BlockSpec(shape, lambda i: (0,) * len(shape))

    out = pl.pallas_call(
        _fused_body,
        out_shape=jax.ShapeDtypeStruct((rows, D), seq_feat.dtype),
        grid=(nblk,),
        in_specs=[
            row_spec,
            row_spec,
            full((4, D)),
            full((2, D)),
            full((2, D)),
            full((1, 2)),
        ],
        out_specs=row_spec,
        compiler_params=pltpu.CompilerParams(
            dimension_semantics=("parallel",),
        ),
        name="attention_fusion",
    )(seq2, msa2, gw, gamma2, beta2, gb)
    return out.reshape(B, S, D)


# speedup vs baseline: 1.1780x; 1.0276x over previous
import jax
import jax.numpy as jnp
from jax.experimental import pallas as pl
from jax.experimental.pallas import tpu as pltpu

_LN_EPS = 1e-5
_BLK = 1024
_N_STEPS = 16


def _prep_weights(gw_ref, gamma_ref, beta_ref, gb_ref):
    g = gamma_ref[...]
    bt = beta_ref[...]
    w = gw_ref[...]
    dws = g[0:1, :] * (w[0:1, :] - w[2:3, :])
    dwm = g[1:2, :] * (w[1:2, :] - w[3:4, :])
    da = jnp.sum(dws, axis=-1, keepdims=True) + jnp.sum(dwm, axis=-1, keepdims=True)
    dcb = (jnp.sum(bt[0:1, :] * (w[0:1, :] - w[2:3, :]), axis=-1, keepdims=True)
           + jnp.sum(bt[1:2, :] * (w[1:2, :] - w[3:4, :]), axis=-1, keepdims=True)
           + (gb_ref[0:1, 0:1] - gb_ref[0:1, 1:2]))
    return dws, dwm, da, dcb


def _blend_block(s, m, dws, dwm, da, dcb):
    n_inv = 1.0 / (2.0 * s.shape[-1])
    row_sum = (jnp.sum(s, axis=-1, keepdims=True)
               + jnp.sum(m, axis=-1, keepdims=True))
    row_sq = (jnp.sum(s * s, axis=-1, keepdims=True)
              + jnp.sum(m * m, axis=-1, keepdims=True))
    mean = row_sum * n_inv
    var = row_sq * n_inv - mean * mean
    rstd = jax.lax.rsqrt(var + _LN_EPS)
    t = (jnp.sum(s * dws, axis=-1, keepdims=True)
         + jnp.sum(m * dwm, axis=-1, keepdims=True))
    dl = (t - mean * da) * rstd + dcb
    w0 = jax.nn.sigmoid(dl)
    return m + w0 * (s - m)


def _pipelined_body(seq_hbm, msa_hbm, gw_ref, gamma_ref, beta_ref, gb_ref,
                    out_hbm, seq_buf, msa_buf, out_buf,
                    seq_sem, msa_sem, out_sem):
    dws, dwm, da, dcb = _prep_weights(gw_ref, gamma_ref, beta_ref, gb_ref)

    def start_in(slot, step):
        pltpu.make_async_copy(seq_hbm.at[pl.ds(step * _BLK, _BLK), :],
                              seq_buf.at[slot], seq_sem.at[slot]).start()
        pltpu.make_async_copy(msa_hbm.at[pl.ds(step * _BLK, _BLK), :],
                              msa_buf.at[slot], msa_sem.at[slot]).start()

    def wait_in(slot):
        pltpu.make_async_copy(seq_hbm.at[pl.ds(0, _BLK), :],
                              seq_buf.at[slot], seq_sem.at[slot]).wait()
        pltpu.make_async_copy(msa_hbm.at[pl.ds(0, _BLK), :],
                              msa_buf.at[slot], msa_sem.at[slot]).wait()

    def start_out(slot, step):
        pltpu.make_async_copy(out_buf.at[slot],
                              out_hbm.at[pl.ds(step * _BLK, _BLK), :],
                              out_sem.at[slot]).start()

    def wait_out(slot):
        pltpu.make_async_copy(out_buf.at[slot],
                              out_hbm.at[pl.ds(0, _BLK), :],
                              out_sem.at[slot]).wait()

    start_in(0, 0)
    for step in range(_N_STEPS):
        cur = step % 2
        if step + 1 < _N_STEPS:
            start_in((step + 1) % 2, step + 1)
        wait_in(cur)
        if step >= 2:
            wait_out(cur)
        out_buf[cur] = _blend_block(seq_buf[cur], msa_buf[cur],
                                    dws, dwm, da, dcb)
        start_out(cur, step)
    wait_out(_N_STEPS % 2)
    wait_out((_N_STEPS + 1) % 2)


@jax.jit
def kernel(seq_feat, msa_feat, ln_gamma, ln_beta, gate_w, gate_b):
    B, S, D = seq_feat.shape
    rows = B * S
    seq2 = seq_feat.reshape(rows, D)
    msa2 = msa_feat.reshape(rows, D)
    gamma2 = ln_gamma.reshape(2, D)
    beta2 = ln_beta.reshape(2, D)
    gw = gate_w.reshape(4, D)
    gb = gate_b.reshape(1, 2)

    vmem_spec = pl.BlockSpec(memory_space=pltpu.VMEM)
    out = pl.pallas_call(
        _pipelined_body,
        out_shape=jax.ShapeDtypeStruct((rows, D), seq_feat.dtype),
        in_specs=[
            pl.BlockSpec(memory_space=pl.ANY),
            pl.BlockSpec(memory_space=pl.ANY),
            vmem_spec,
            vmem_spec,
            vmem_spec,
            vmem_spec,
        ],
        out_specs=pl.BlockSpec(memory_space=pl.ANY),
        scratch_shapes=[
            pltpu.VMEM((2, _BLK, D), jnp.float32),
            pltpu.VMEM((2, _BLK, D), jnp.float32),
            pltpu.VMEM((2, _BLK, D), jnp.float32),
            pltpu.SemaphoreType.DMA((2,)),
            pltpu.SemaphoreType.DMA((2,)),
            pltpu.SemaphoreType.DMA((2,)),
        ],
        compiler_params=pltpu.CompilerParams(
            vmem_limit_bytes=24 * 1024 * 1024,
        ),
        name="attention_fusion",
    )(seq2, msa2, gw, gamma2, beta2, gb)
    return out.reshape(B, S, D)
